# baseline (speedup 1.0000x reference)
import jax
import jax.numpy as jnp
from jax.experimental import pallas as pl

B = 256
J = 17
S = 96 * 72
TOPK = 8
SB = 8
RB = SB * J


def _sums_body(x_ref, y_ref, o_ref):
    d = x_ref[...] - y_ref[...]
    o_ref[...] = jnp.sum(d * d, axis=1, keepdims=True) * (0.5 / S)


def _mine_body(l_ref, o_ref):
    l = l_ref[...]
    jidx = jax.lax.broadcasted_iota(jnp.int32, (J, B), 0)
    rank = jnp.zeros((J, B), jnp.int32)
    for k in range(J):
        lk = l[k:k + 1, :]
        gt = (lk > l) | ((lk == l) & (k < jidx))
        rank = rank + gt.astype(jnp.int32)
    topsum = jnp.sum(jnp.where(rank < TOPK, l, 0.0))
    o_ref[...] = topsum[None, None] * (1.0 / (TOPK * B))


def kernel(output, target):
    x = output.reshape(B * J, S)
    y = target.reshape(B * J, S)
    sums = pl.pallas_call(
        _sums_body,
        grid=(B * J // RB,),
        in_specs=[
            pl.BlockSpec((RB, S), lambda i: (i, 0)),
            pl.BlockSpec((RB, S), lambda i: (i, 0)),
        ],
        out_specs=pl.BlockSpec((RB, 1), lambda i: (i, 0)),
        out_shape=jax.ShapeDtypeStruct((B * J, 1), jnp.float32),
    )(x, y)
    lt = sums.reshape(B, J).T
    out = pl.pallas_call(
        _mine_body,
        out_shape=jax.ShapeDtypeStruct((1, 1), jnp.float32),
    )(lt)
    return out[0, 0]


# speedup vs baseline: 1.4230x; 1.4230x over previous
import jax
import jax.numpy as jnp
from jax.experimental import pallas as pl

B = 256
J = 17
H = 96
W = 72
S = H * W
TOPK = 8
BB = 8


def _sums_body(x_ref, y_ref, o_ref):
    d = x_ref[...] - y_ref[...]
    o_ref[...] = jnp.sum(d * d, axis=(2, 3)) * (0.5 / S)


def _mine_body(l_ref, o_ref):
    l = l_ref[...]
    jidx = jax.lax.broadcasted_iota(jnp.int32, (J, B), 0)
    rank = jnp.zeros((J, B), jnp.int32)
    for k in range(J):
        lk = l[k:k + 1, :]
        gt = (lk > l) | ((lk == l) & (k < jidx))
        rank = rank + gt.astype(jnp.int32)
    topsum = jnp.sum(jnp.where(rank < TOPK, l, 0.0))
    o_ref[...] = topsum[None, None] * (1.0 / (TOPK * B))


def kernel(output, target):
    losses = pl.pallas_call(
        _sums_body,
        grid=(B // BB,),
        in_specs=[
            pl.BlockSpec((BB, J, H, W), lambda i: (i, 0, 0, 0)),
            pl.BlockSpec((BB, J, H, W), lambda i: (i, 0, 0, 0)),
        ],
        out_specs=pl.BlockSpec((BB, J), lambda i: (i, 0)),
        out_shape=jax.ShapeDtypeStruct((B, J), jnp.float32),
    )(output, target)
    out = pl.pallas_call(
        _mine_body,
        out_shape=jax.ShapeDtypeStruct((1, 1), jnp.float32),
    )(losses.T)
    return out[0, 0]


# speedup vs baseline: 2.3411x; 1.6452x over previous
import jax
import jax.numpy as jnp
from jax.experimental import pallas as pl

B = 256
J = 17
S = 96 * 72
TOPK = 8
BB = 8


def _sums_body(x_ref, y_ref, o_ref):
    d = x_ref[...] - y_ref[...]
    d2 = d * d
    for j in range(J):
        s = jnp.sum(d2[:, j * S:(j + 1) * S], axis=1, keepdims=True)
        o_ref[:, j:j + 1] = s * (0.5 / S)


def _mine_body(l_ref, o_ref):
    l = l_ref[...]
    jidx = jax.lax.broadcasted_iota(jnp.int32, (J, B), 0)
    rank = jnp.zeros((J, B), jnp.int32)
    for k in range(J):
        lk = l[k:k + 1, :]
        gt = (lk > l) | ((lk == l) & (k < jidx))
        rank = rank + gt.astype(jnp.int32)
    topsum = jnp.sum(jnp.where(rank < TOPK, l, 0.0))
    o_ref[...] = topsum[None, None] * (1.0 / (TOPK * B))


def kernel(output, target):
    x = output.reshape(B, J * S)
    y = target.reshape(B, J * S)
    losses = pl.pallas_call(
        _sums_body,
        grid=(B // BB,),
        in_specs=[
            pl.BlockSpec((BB, J * S), lambda i: (i, 0)),
            pl.BlockSpec((BB, J * S), lambda i: (i, 0)),
        ],
        out_specs=pl.BlockSpec((BB, J), lambda i: (i, 0)),
        out_shape=jax.ShapeDtypeStruct((B, J), jnp.float32),
    )(x, y)
    out = pl.pallas_call(
        _mine_body,
        out_shape=jax.ShapeDtypeStruct((1, 1), jnp.float32),
    )(losses.T)
    return out[0, 0]


# speedup vs baseline: 2.3477x; 1.0028x over previous
import jax
import jax.numpy as jnp
from jax.experimental import pallas as pl

B = 256
J = 17
S = 96 * 72
TOPK = 8
BB = 8


def _sums_body(*refs):
    xs, ys, o_ref = refs[:J], refs[J:2 * J], refs[2 * J]
    for j in range(J):
        d = xs[j][...] - ys[j][...]
        s = jnp.sum(d * d, axis=1, keepdims=True)
        o_ref[:, j:j + 1] = s * (0.5 / S)


def _mine_body(l_ref, o_ref):
    l = l_ref[...]
    jidx = jax.lax.broadcasted_iota(jnp.int32, (J, B), 0)
    rank = jnp.zeros((J, B), jnp.int32)
    for k in range(J):
        lk = l[k:k + 1, :]
        gt = (lk > l) | ((lk == l) & (k < jidx))
        rank = rank + gt.astype(jnp.int32)
    topsum = jnp.sum(jnp.where(rank < TOPK, l, 0.0))
    o_ref[...] = topsum[None, None] * (1.0 / (TOPK * B))


def kernel(output, target):
    x = output.reshape(B, J * S)
    y = target.reshape(B, J * S)
    specs = [pl.BlockSpec((BB, S), lambda i, j=j: (i, j)) for j in range(J)]
    losses = pl.pallas_call(
        _sums_body,
        grid=(B // BB,),
        in_specs=specs + specs,
        out_specs=pl.BlockSpec((BB, J), lambda i: (i, 0)),
        out_shape=jax.ShapeDtypeStruct((B, J), jnp.float32),
    )(*([x] * J + [y] * J))
    out = pl.pallas_call(
        _mine_body,
        out_shape=jax.ShapeDtypeStruct((1, 1), jnp.float32),
    )(losses.T)
    return out[0, 0]
